# baseline (device time: 31775 ns/iter reference)
import jax
import jax.numpy as jnp
from jax import lax
from jax.experimental import pallas as pl
from jax.experimental.pallas import tpu as pltpu

N_DEV = 16
MASKS = [1, 2, 4, 8]


def kernel(x, Wq, K_ext, V_ext, Wo):
    B, Sq, D = x.shape
    _, Hd = Wq.shape
    _, Skv, Hq, Dh = K_ext.shape
    Hloc = Hd // Dh
    T = B * Sq
    HL = T // 4
    HC = D // 2

    my = lax.axis_index("i")
    K_loc = lax.dynamic_slice_in_dim(K_ext, my * Hloc, Hloc, axis=2)
    K_loc = K_loc.reshape(B, Skv, Hloc * Dh)
    V_loc = lax.dynamic_slice_in_dim(V_ext, my * Hloc, Hloc, axis=2)
    V_loc = V_loc.reshape(B, Skv, Hloc * Dh)

    def body(x_ref, wq_ref, k_ref, v_ref, wo_ref, acc_ref,
             sb0, rb0, ss, sr, send_sems, recv_sems):
        my_pos = lax.axis_index("i")
        pending = []

        barrier_sem = pltpu.get_barrier_semaphore()
        for m in MASKS:
            pl.semaphore_signal(
                barrier_sem, inc=1,
                device_id=(jnp.bitwise_xor(my_pos, m),),
                device_id_type=pl.DeviceIdType.MESH)

        qi = lax.broadcasted_iota(jnp.int32, (Sq, Skv), 0)
        ki = lax.broadcasted_iota(jnp.int32, (Sq, Skv), 1)
        mask = (jnp.abs(qi - ki) <= 128) | (ki < 32) | (qi < 32)

        def slab(b):
            qfull = jnp.dot(x_ref[b], wq_ref[...],
                            preferred_element_type=jnp.float32)
            acc = jnp.zeros((Sq, D), jnp.float32)
            for h in range(Hloc):
                q = qfull[:, h * Dh:(h + 1) * Dh]
                k = k_ref[b][:, h * Dh:(h + 1) * Dh]
                v = v_ref[b][:, h * Dh:(h + 1) * Dh]
                s = lax.dot_general(
                    q, k, (((1,), (1,)), ((), ())),
                    preferred_element_type=jnp.float32) * 0.125
                s = jnp.where(mask, s, -1e9)
                mx = jnp.max(s, axis=1, keepdims=True)
                w = jnp.exp(s - mx)
                w = w / jnp.sum(w, axis=1, keepdims=True)
                ctx = jnp.dot(w, v, preferred_element_type=jnp.float32)
                acc = acc + jnp.dot(
                    ctx, wo_ref[h * Dh:(h + 1) * Dh, :],
                    preferred_element_type=jnp.float32)
            acc_ref[b * Sq:(b + 1) * Sq, :] = acc

        hl0 = T // 2
        bit0 = (jnp.bitwise_and(my_pos, 1) > 0).astype(jnp.int32)
        bit4 = (jnp.bitwise_and(my_pos, 4) > 0).astype(jnp.int32)
        rdma0 = pltpu.make_async_remote_copy(
            src_ref=sb0, dst_ref=rb0,
            send_sem=send_sems.at[0], recv_sem=recv_sems.at[0],
            device_id=(jnp.bitwise_xor(my_pos, 1),),
            device_id_type=pl.DeviceIdType.MESH,
        )
        pending.append(rdma0)

        @pl.when(bit0 == 1)
        def _():
            slab(0)
            pl.semaphore_wait(barrier_sem, len(MASKS))
            sb0[...] = acc_ref[0:hl0, :].astype(jnp.bfloat16)
            rdma0.start()
            slab(1)

        @pl.when(bit0 == 0)
        def _():
            slab(1)
            pl.semaphore_wait(barrier_sem, len(MASKS))
            sb0[...] = acc_ref[hl0:T, :].astype(jnp.bfloat16)
            rdma0.start()
            slab(0)

        slab_base = pl.multiple_of(hl0 * bit0, hl0)
        o1_send = pl.multiple_of(slab_base + HL * (1 - bit4), HL)
        o = pl.multiple_of(slab_base + HL * bit4, HL)
        r4_recv = o1_send
        sib_base = pl.multiple_of(hl0 * (1 - bit0), hl0)
        recv_a = pl.multiple_of(sib_base + HL * bit4, HL)
        recv_b = pl.multiple_of(sib_base + HL * (1 - bit4), HL)

        rdma0.wait_recv()
        rb_lo = pl.multiple_of(HL * (1 - bit4), HL)
        rb_hi = pl.multiple_of(HL * bit4, HL)
        acc_ref[pl.ds(o1_send, HL), :] = (
            acc_ref[pl.ds(o1_send, HL), :]
            + rb0[pl.ds(rb_lo, HL), :].astype(jnp.float32))

        sem_idx = [1]

        def cs(idx, rows, c0, m):
            ss[idx] = acc_ref[pl.ds(rows, HL), c0:c0 + HC].astype(
                jnp.bfloat16)
            return send_to(idx, idx, m)

        def send_to(src_idx, dst_idx, m):
            slot = sem_idx[0]
            sem_idx[0] += 1
            rdma = pltpu.make_async_remote_copy(
                src_ref=ss.at[src_idx], dst_ref=sr.at[dst_idx],
                send_sem=send_sems.at[slot], recv_sem=recv_sems.at[slot],
                device_id=(jnp.bitwise_xor(my_pos, m),),
                device_id_type=pl.DeviceIdType.MESH,
            )
            rdma.start()
            pending.append(rdma)
            return rdma

        def add(idx, rows, c0, rdma):
            rdma.wait_recv()
            acc_ref[pl.ds(rows, HL), c0:c0 + HC] = (
                acc_ref[pl.ds(rows, HL), c0:c0 + HC]
                + sr[idx].astype(jnp.float32))

        def store(idx, rows, c0, rdma):
            rdma.wait_recv()
            acc_ref[pl.ds(rows, HL), c0:c0 + HC] = (
                sr[idx].astype(jnp.float32))

        A, Bc = 0, HC
        r1a, r1b, xga, xgb = 0, 1, 2, 3
        xg2a, xg8a, xg10a = 2, 4, 5
        xg2b, xg8b, xg10b = 3, 6, 7
        r45a, r45b, r5ba_s, r5bb_s = 4, 5, 6, 7
        r4a_r, r4b_r, r5aa_r, r5ab_r, r5ba_r, r5bb_r = 8, 9, 10, 11, 12, 13

        d_r1a = cs(r1a, o1_send, A, 4)
        d_r1b = cs(r1b, o1_send, Bc, 4)
        acc_ref[pl.ds(o, HL), :] = (
            acc_ref[pl.ds(o, HL), :]
            + rb0[pl.ds(rb_hi, HL), :].astype(jnp.float32))
        add(r1a, o, A, d_r1a)

        xga_idx = xga
        ss[xga_idx] = acc_ref[pl.ds(o, HL), A:A + HC].astype(jnp.bfloat16)
        d_g2a = send_to(xga_idx, xg2a, 2)
        d_g8a = send_to(xga_idx, xg8a, 8)
        d_g10a = send_to(xga_idx, xg10a, 10)
        add(r1b, o, Bc, d_r1b)
        ss[xgb] = acc_ref[pl.ds(o, HL), Bc:Bc + HC].astype(jnp.bfloat16)
        d_g2b = send_to(xgb, xg2b, 2)
        d_g8b = send_to(xgb, xg8b, 8)
        d_g10b = send_to(xgb, xg10b, 10)
        add(xg2a, o, A, d_g2a)
        add(xg2b, o, Bc, d_g2b)
        add(xg8a, o, A, d_g8a)
        add(xg8b, o, Bc, d_g8b)
        add(xg10a, o, A, d_g10a)
        ss[r45a] = acc_ref[pl.ds(o, HL), A:A + HC].astype(jnp.bfloat16)
        d_r4a = send_to(r45a, r4a_r, 4)
        d_r5aa = send_to(r45a, r5aa_r, 1)
        add(xg10b, o, Bc, d_g10b)
        ss[r45b] = acc_ref[pl.ds(o, HL), Bc:Bc + HC].astype(jnp.bfloat16)
        d_r4b = send_to(r45b, r4b_r, 4)
        d_r5ab = send_to(r45b, r5ab_r, 1)
        store(r4a_r, r4_recv, A, d_r4a)
        ss[r5ba_s] = acc_ref[pl.ds(r4_recv, HL), A:A + HC].astype(
            jnp.bfloat16)
        d_r5ba = send_to(r5ba_s, r5ba_r, 1)
        store(r4b_r, r4_recv, Bc, d_r4b)
        ss[r5bb_s] = acc_ref[pl.ds(r4_recv, HL), Bc:Bc + HC].astype(
            jnp.bfloat16)
        d_r5bb = send_to(r5bb_s, r5bb_r, 1)
        store(r5aa_r, recv_a, A, d_r5aa)
        store(r5ab_r, recv_a, Bc, d_r5ab)
        store(r5ba_r, recv_b, A, d_r5ba)
        store(r5bb_r, recv_b, Bc, d_r5bb)

        for rdma in pending:
            rdma.wait_send()

    out = pl.pallas_call(
        body,
        out_shape=jax.ShapeDtypeStruct((T, D), jnp.float32),
        in_specs=[pl.BlockSpec(memory_space=pltpu.VMEM)] * 5,
        out_specs=pl.BlockSpec(memory_space=pltpu.VMEM),
        scratch_shapes=[
            pltpu.VMEM((T // 2, D), jnp.bfloat16),
            pltpu.VMEM((T // 2, D), jnp.bfloat16),
            pltpu.VMEM((8, HL, HC), jnp.bfloat16),
            pltpu.VMEM((14, HL, HC), jnp.bfloat16),
            pltpu.SemaphoreType.DMA((15,)),
            pltpu.SemaphoreType.DMA((15,)),
        ],
        compiler_params=pltpu.CompilerParams(collective_id=0),
    )(x, Wq, K_loc, V_loc, Wo)
    return out.reshape(B, Sq, D)


# device time: 30875 ns/iter; 1.0291x vs baseline; 1.0291x over previous
import jax
import jax.numpy as jnp
from jax import lax
from jax.experimental import pallas as pl
from jax.experimental.pallas import tpu as pltpu

N_DEV = 16
MASKS = [1, 2, 4, 8]


def kernel(x, Wq, K_ext, V_ext, Wo):
    B, Sq, D = x.shape
    _, Hd = Wq.shape
    _, Skv, Hq, Dh = K_ext.shape
    Hloc = Hd // Dh
    T = B * Sq
    HL = T // 4
    HC = D // 2

    my = lax.axis_index("i")
    K_loc = lax.dynamic_slice_in_dim(K_ext, my * Hloc, Hloc, axis=2)
    K_loc = K_loc.reshape(B, Skv, Hloc * Dh)
    V_loc = lax.dynamic_slice_in_dim(V_ext, my * Hloc, Hloc, axis=2)
    V_loc = V_loc.reshape(B, Skv, Hloc * Dh)

    def body(x_ref, wq_ref, k_ref, v_ref, wo_ref, acc_ref,
             sb0, rb0, ss, sr, send_sems, recv_sems):
        my_pos = lax.axis_index("i")
        pending = []

        barrier_sem = pltpu.get_barrier_semaphore()
        for m in MASKS:
            pl.semaphore_signal(
                barrier_sem, inc=1,
                device_id=(jnp.bitwise_xor(my_pos, m),),
                device_id_type=pl.DeviceIdType.MESH)

        qi = lax.broadcasted_iota(jnp.int32, (Sq, Skv), 0)
        ki = lax.broadcasted_iota(jnp.int32, (Sq, Skv), 1)
        mask = (jnp.abs(qi - ki) <= 128) | (ki < 32) | (qi < 32)

        def slab(b):
            qfull = jnp.dot(x_ref[b], wq_ref[...],
                            preferred_element_type=jnp.float32)
            acc = jnp.zeros((Sq, D), jnp.float32)
            for h in range(Hloc):
                q = qfull[:, h * Dh:(h + 1) * Dh]
                k = k_ref[b][:, h * Dh:(h + 1) * Dh]
                v = v_ref[b][:, h * Dh:(h + 1) * Dh]
                s = lax.dot_general(
                    q, k, (((1,), (1,)), ((), ())),
                    preferred_element_type=jnp.float32) * 0.125
                s = jnp.where(mask, s, -1e9)
                mx = jnp.max(s, axis=1, keepdims=True)
                w = jnp.exp(s - mx)
                w = w / jnp.sum(w, axis=1, keepdims=True)
                ctx = jnp.dot(w, v, preferred_element_type=jnp.float32)
                acc = acc + jnp.dot(
                    ctx, wo_ref[h * Dh:(h + 1) * Dh, :],
                    preferred_element_type=jnp.float32)
            acc_ref[b] = acc

        hl0 = T // 2
        bit0 = (jnp.bitwise_and(my_pos, 1) > 0).astype(jnp.int32)
        bit4 = (jnp.bitwise_and(my_pos, 4) > 0).astype(jnp.int32)
        rdma0 = pltpu.make_async_remote_copy(
            src_ref=sb0, dst_ref=rb0,
            send_sem=send_sems.at[0], recv_sem=recv_sems.at[0],
            device_id=(jnp.bitwise_xor(my_pos, 1),),
            device_id_type=pl.DeviceIdType.MESH,
        )
        pending.append(rdma0)

        @pl.when(bit0 == 1)
        def _():
            slab(0)
            pl.semaphore_wait(barrier_sem, len(MASKS))
            sb0[...] = acc_ref[0].astype(jnp.bfloat16)
            rdma0.start()
            slab(1)

        @pl.when(bit0 == 0)
        def _():
            slab(1)
            pl.semaphore_wait(barrier_sem, len(MASKS))
            sb0[...] = acc_ref[1].astype(jnp.bfloat16)
            rdma0.start()
            slab(0)

        rdma0.wait_recv()
        own = bit0
        sib = 1 - bit0
        acc_ref[pl.ds(own, 1), :, :] = (
            acc_ref[pl.ds(own, 1), :, :]
            + rb0[...].astype(jnp.float32)[None])

        r_keep = pl.multiple_of(HL * bit4, HL)
        r_send = pl.multiple_of(HL * (1 - bit4), HL)

        def cs(idx, slb, row, c0, m):
            ss[idx] = acc_ref[pl.ds(slb, 1), pl.ds(row, HL),
                              c0:c0 + HC].astype(jnp.bfloat16)
            rdma = pltpu.make_async_remote_copy(
                src_ref=ss.at[idx], dst_ref=sr.at[idx],
                send_sem=send_sems.at[idx + 1],
                recv_sem=recv_sems.at[idx + 1],
                device_id=(jnp.bitwise_xor(my_pos, m),),
                device_id_type=pl.DeviceIdType.MESH,
            )
            rdma.start()
            pending.append(rdma)
            return rdma

        def add(idx, slb, row, c0, rdma):
            rdma.wait_recv()
            acc_ref[pl.ds(slb, 1), pl.ds(row, HL), c0:c0 + HC] = (
                acc_ref[pl.ds(slb, 1), pl.ds(row, HL), c0:c0 + HC]
                + sr[idx].astype(jnp.float32))

        def store(idx, slb, row, c0, rdma):
            rdma.wait_recv()
            acc_ref[pl.ds(slb, 1), pl.ds(row, HL), c0:c0 + HC] = (
                sr[idx].astype(jnp.float32))

        A, Bc = 0, HC
        r1a, r1b, x2a, x2b, x3a, x3b = 0, 1, 2, 3, 4, 5
        r4a, r4b, r5aa, r5ab, r5ba, r5bb = 6, 7, 8, 9, 10, 11

        d_r1a = cs(r1a, own, r_send, A, 4)
        d_r1b = cs(r1b, own, r_send, Bc, 4)
        add(r1a, own, r_keep, A, d_r1a)
        d_x2a = cs(x2a, own, r_keep, A, 2)
        add(r1b, own, r_keep, Bc, d_r1b)
        d_x2b = cs(x2b, own, r_keep, Bc, 2)
        add(x2a, own, r_keep, A, d_x2a)
        d_x3a = cs(x3a, own, r_keep, A, 8)
        add(x2b, own, r_keep, Bc, d_x2b)
        d_x3b = cs(x3b, own, r_keep, Bc, 8)
        add(x3a, own, r_keep, A, d_x3a)
        d_r4a = cs(r4a, own, r_keep, A, 4)
        d_r5aa = cs(r5aa, own, r_keep, A, 1)
        add(x3b, own, r_keep, Bc, d_x3b)
        d_r4b = cs(r4b, own, r_keep, Bc, 4)
        d_r5ab = cs(r5ab, own, r_keep, Bc, 1)
        store(r4a, own, r_send, A, d_r4a)
        d_r5ba = cs(r5ba, own, r_send, A, 1)
        store(r4b, own, r_send, Bc, d_r4b)
        d_r5bb = cs(r5bb, own, r_send, Bc, 1)
        store(r5aa, sib, r_keep, A, d_r5aa)
        store(r5ab, sib, r_keep, Bc, d_r5ab)
        store(r5ba, sib, r_send, A, d_r5ba)
        store(r5bb, sib, r_send, Bc, d_r5bb)

        for rdma in pending:
            rdma.wait_send()

    out = pl.pallas_call(
        body,
        out_shape=jax.ShapeDtypeStruct((B, Sq, D), jnp.float32),
        in_specs=[pl.BlockSpec(memory_space=pltpu.VMEM)] * 5,
        out_specs=pl.BlockSpec(memory_space=pltpu.VMEM),
        scratch_shapes=[
            pltpu.VMEM((T // 2, D), jnp.bfloat16),
            pltpu.VMEM((T // 2, D), jnp.bfloat16),
            pltpu.VMEM((12, 1, HL, HC), jnp.bfloat16),
            pltpu.VMEM((12, 1, HL, HC), jnp.bfloat16),
            pltpu.SemaphoreType.DMA((13,)),
            pltpu.SemaphoreType.DMA((13,)),
        ],
        compiler_params=pltpu.CompilerParams(collective_id=0),
    )(x, Wq, K_loc, V_loc, Wo)
    return out
